# Initial kernel scaffold; baseline (speedup 1.0000x reference)
import jax
import jax.numpy as jnp
from jax.experimental import pallas as pl


def kernel(node_latents, mesh_edge_latents, senders, receivers, eW1, eb1, eW2, eb2, eg, ebt, nW1, nb1, nW2, nb2, ng, nbt):
    raise NotImplementedError("write your pallas kernel here")


# speedup vs baseline: 2.5103x; 2.5103x over previous
import functools

import jax
import jax.numpy as jnp
from jax import lax
from jax.experimental import pallas as pl
from jax.experimental.pallas import tpu as pltpu
from jax.experimental.pallas import tpu_sc as plsc

N = 10000
E = 320000
D = 128

NC = 2
NS = 16
NW = NC * NS

EPT = E // NW
KC = 80
NCH = EPT // KC

EPC = E // NC
EPT_S = EPC // NS
CPT_S = EPT_S // KC
NPAD = 10240
ROWS_PT = NPAD // NS

_MESH = plsc.VectorSubcoreMesh(core_axis_name="c", subcore_axis_name="s")


def _proj_body(node_ref, wa_ref, wb_ref, sp_ref, rp_ref):
    n = node_ref[...]
    sp_ref[...] = jnp.dot(n, wa_ref[...], preferred_element_type=jnp.float32)
    rp_ref[...] = jnp.dot(n, wb_ref[...], preferred_element_type=jnp.float32)


def _project(node, wa, wb):
    bn = 1000
    grid = (N // bn,)
    return pl.pallas_call(
        _proj_body,
        grid=grid,
        in_specs=[
            pl.BlockSpec((bn, D), lambda i: (i, 0)),
            pl.BlockSpec((D, D), lambda i: (0, 0)),
            pl.BlockSpec((D, D), lambda i: (0, 0)),
        ],
        out_specs=[
            pl.BlockSpec((bn, D), lambda i: (i, 0)),
            pl.BlockSpec((bn, D), lambda i: (i, 0)),
        ],
        out_shape=[
            jax.ShapeDtypeStruct((N, D), jnp.float32),
            jax.ShapeDtypeStruct((N, D), jnp.float32),
        ],
    )(node, wa, wb)


@functools.partial(
    pl.kernel,
    out_type=jax.ShapeDtypeStruct((E, D), jnp.float32),
    mesh=_MESH,
    scratch_types=[
        pltpu.VMEM((KC,), jnp.int32),
        pltpu.VMEM((KC,), jnp.int32),
        pltpu.VMEM((KC, D), jnp.float32),
        pltpu.VMEM((KC, D), jnp.float32),
        pltpu.SemaphoreType.DMA,
        pltpu.SemaphoreType.DMA,
    ],
)
def _gather_sum(sp_hbm, rp_hbm, snd_hbm, rcv_hbm, out_hbm,
                idx_s, idx_r, rows_s, rows_r, sem_s, sem_r):
    wid = lax.axis_index("s") * NC + lax.axis_index("c")
    base = wid * EPT

    def chunk(i, carry):
        off = base + i * KC
        pltpu.sync_copy(snd_hbm.at[pl.ds(off, KC)], idx_s)
        pltpu.sync_copy(rcv_hbm.at[pl.ds(off, KC)], idx_r)
        cp_s = pltpu.async_copy(sp_hbm.at[idx_s], rows_s, sem_s)
        cp_r = pltpu.async_copy(rp_hbm.at[idx_r], rows_r, sem_r)
        cp_s.wait()
        cp_r.wait()

        def add_row(j, c2):
            for l in range(D // 16):
                sl = pl.ds(l * 16, 16)
                rows_s[j, sl] = rows_s[j, sl] + rows_r[j, sl]
            return c2

        lax.fori_loop(0, KC, add_row, 0, unroll=2)
        pltpu.sync_copy(rows_s, out_hbm.at[pl.ds(off, KC)])
        return carry

    lax.fori_loop(0, NCH, chunk, 0)


def _edge_body(gs_ref, el_ref, w1_ref, b1_ref, w2_ref, b2_ref, g_ref, bt_ref,
               me_ref, out_ref):
    el = el_ref[...]
    h1 = gs_ref[...] + jnp.dot(el, w1_ref[...], preferred_element_type=jnp.float32)
    h1 = jnp.maximum(h1 + b1_ref[...], 0.0)
    h2 = jnp.dot(h1, w2_ref[...], preferred_element_type=jnp.float32)
    h2 = jnp.maximum(h2 + b2_ref[...], 0.0)
    mu = jnp.mean(h2, axis=-1, keepdims=True)
    var = jnp.mean((h2 - mu) ** 2, axis=-1, keepdims=True)
    ln = (h2 - mu) * lax.rsqrt(var + 1e-5) * g_ref[...] + bt_ref[...]
    me_ref[...] = ln
    out_ref[...] = ln + el


def _edge_mlp(gsum, edge_latents, w1c, b1, w2, b2, g, bt):
    be = 2000
    grid = (E // be,)
    row = lambda i: (i, 0)
    full = lambda i: (0, 0)
    return pl.pallas_call(
        _edge_body,
        grid=grid,
        in_specs=[
            pl.BlockSpec((be, D), row),
            pl.BlockSpec((be, D), row),
            pl.BlockSpec((D, D), full),
            pl.BlockSpec((1, D), full),
            pl.BlockSpec((D, D), full),
            pl.BlockSpec((1, D), full),
            pl.BlockSpec((1, D), full),
            pl.BlockSpec((1, D), full),
        ],
        out_specs=[
            pl.BlockSpec((be, D), row),
            pl.BlockSpec((be, D), row),
        ],
        out_shape=[
            jax.ShapeDtypeStruct((E, D), jnp.float32),
            jax.ShapeDtypeStruct((E, D), jnp.float32),
        ],
    )(gsum, edge_latents, w1c, b1, w2, b2, g, bt)


@functools.partial(
    pl.kernel,
    out_type=jax.ShapeDtypeStruct((NC, NPAD, D), jnp.float32),
    mesh=_MESH,
    scratch_types=[
        pltpu.VMEM((KC,), jnp.int32),
        pltpu.VMEM((KC, D), jnp.float32),
        pltpu.VMEM_SHARED((NPAD, D), jnp.float32),
        pltpu.SemaphoreType.DMA,
    ],
)
def _scatter_add(me_hbm, rcv_hbm, zeros_hbm, out_hbm, idx_v, rows_v, acc, sem):
    c = lax.axis_index("c")
    s = lax.axis_index("s")
    rbase = s * ROWS_PT
    pltpu.sync_copy(zeros_hbm.at[pl.ds(rbase, ROWS_PT)],
                    acc.at[pl.ds(rbase, ROWS_PT)])
    plsc.subcore_barrier()

    base = c * EPC + s * EPT_S

    def chunk(i, carry):
        off = base + i * KC
        pltpu.sync_copy(rcv_hbm.at[pl.ds(off, KC)], idx_v)
        pltpu.sync_copy(me_hbm.at[pl.ds(off, KC)], rows_v)
        pltpu.sync_copy(rows_v, acc.at[idx_v], add=True)
        return carry

    lax.fori_loop(0, CPT_S, chunk, 0)
    plsc.subcore_barrier()
    pltpu.sync_copy(acc.at[pl.ds(rbase, ROWS_PT)],
                    out_hbm.at[c, pl.ds(rbase, ROWS_PT)])


def _node_body(nd_ref, p_ref, w1a_ref, w1b_ref, b1_ref, w2_ref, b2_ref,
               g_ref, bt_ref, out_ref):
    nd = nd_ref[...]
    aggr = p_ref[0] + p_ref[1]
    h1 = (jnp.dot(nd, w1a_ref[...], preferred_element_type=jnp.float32)
          + jnp.dot(aggr, w1b_ref[...], preferred_element_type=jnp.float32))
    h1 = jnp.maximum(h1 + b1_ref[...], 0.0)
    h2 = jnp.dot(h1, w2_ref[...], preferred_element_type=jnp.float32)
    h2 = jnp.maximum(h2 + b2_ref[...], 0.0)
    mu = jnp.mean(h2, axis=-1, keepdims=True)
    var = jnp.mean((h2 - mu) ** 2, axis=-1, keepdims=True)
    ln = (h2 - mu) * lax.rsqrt(var + 1e-5) * g_ref[...] + bt_ref[...]
    out_ref[...] = ln + nd


def _node_mlp(node, partials, w1a, w1b, b1, w2, b2, g, bt):
    bn = 1000
    grid = (N // bn,)
    row = lambda i: (i, 0)
    full = lambda i: (0, 0)
    return pl.pallas_call(
        _node_body,
        grid=grid,
        in_specs=[
            pl.BlockSpec((bn, D), row),
            pl.BlockSpec((NC, bn, D), lambda i: (0, i, 0)),
            pl.BlockSpec((D, D), full),
            pl.BlockSpec((D, D), full),
            pl.BlockSpec((1, D), full),
            pl.BlockSpec((D, D), full),
            pl.BlockSpec((1, D), full),
            pl.BlockSpec((1, D), full),
            pl.BlockSpec((1, D), full),
        ],
        out_specs=pl.BlockSpec((bn, D), row),
        out_shape=jax.ShapeDtypeStruct((N, D), jnp.float32),
    )(node, partials, w1a, w1b, b1, w2, b2, g, bt)


def kernel(node_latents, mesh_edge_latents, senders, receivers,
           eW1, eb1, eW2, eb2, eg, ebt,
           nW1, nb1, nW2, nb2, ng, nbt):
    senders = senders.astype(jnp.int32)
    receivers = receivers.astype(jnp.int32)

    sp, rp = _project(node_latents, eW1[:D], eW1[D:2 * D])
    gsum = _gather_sum(sp, rp, senders, receivers)
    me, new_edges = _edge_mlp(
        gsum, mesh_edge_latents, eW1[2 * D:],
        eb1.reshape(1, D), eW2, eb2.reshape(1, D),
        eg.reshape(1, D), ebt.reshape(1, D))
    zeros = jnp.zeros((NPAD, D), jnp.float32)
    partials = _scatter_add(me, receivers, zeros)
    new_nodes = _node_mlp(
        node_latents, partials, nW1[:D], nW1[D:],
        nb1.reshape(1, D), nW2, nb2.reshape(1, D),
        ng.reshape(1, D), nbt.reshape(1, D))
    return (new_nodes, new_edges)


# speedup vs baseline: 3.7423x; 1.4908x over previous
import functools

import jax
import jax.numpy as jnp
from jax import lax
from jax.experimental import pallas as pl
from jax.experimental.pallas import tpu as pltpu
from jax.experimental.pallas import tpu_sc as plsc

N = 10000
E = 320000
D = 128

NC = 2
NS = 16
NW = NC * NS

EPT = E // NW
KC = 80
NCH = EPT // KC

NCHP = 128
NPAD = 10240
ROWS_PT = NPAD // NS
NBUF = 2

_MESH = plsc.VectorSubcoreMesh(core_axis_name="c", subcore_axis_name="s")


def _proj_body(node_ref, wa_ref, wb_ref, sp_ref, rp_ref):
    n = node_ref[...]
    sp_ref[...] = jnp.dot(n, wa_ref[...], preferred_element_type=jnp.float32)
    rp_ref[...] = jnp.dot(n, wb_ref[...], preferred_element_type=jnp.float32)


def _project(node, wa, wb):
    bn = 1000
    grid = (N // bn,)
    return pl.pallas_call(
        _proj_body,
        grid=grid,
        in_specs=[
            pl.BlockSpec((bn, D), lambda i: (i, 0)),
            pl.BlockSpec((D, D), lambda i: (0, 0)),
            pl.BlockSpec((D, D), lambda i: (0, 0)),
        ],
        out_specs=[
            pl.BlockSpec((bn, D), lambda i: (i, 0)),
            pl.BlockSpec((bn, D), lambda i: (i, 0)),
        ],
        out_shape=[
            jax.ShapeDtypeStruct((N, D), jnp.float32),
            jax.ShapeDtypeStruct((N, D), jnp.float32),
        ],
    )(node, wa, wb)


@functools.partial(
    pl.kernel,
    out_type=jax.ShapeDtypeStruct((E, D), jnp.float32),
    mesh=_MESH,
    scratch_types=[
        pltpu.VMEM((NCHP, KC), jnp.int32),
        pltpu.VMEM((NCHP, KC), jnp.int32),
        pltpu.VMEM((NBUF, KC, D), jnp.float32),
        pltpu.VMEM((NBUF, KC, D), jnp.float32),
        pltpu.VMEM((NBUF, KC, D), jnp.float32),
        pltpu.SemaphoreType.DMA,
        pltpu.SemaphoreType.DMA,
        pltpu.SemaphoreType.DMA,
        pltpu.SemaphoreType.DMA,
        pltpu.SemaphoreType.DMA,
    ],
)
def _gather_sum(sp_hbm, rp_hbm, snd3_hbm, rcv3_hbm, out_hbm,
                idx_s2, idx_r2, rows_s, rows_r, rows_o,
                gsem0, gsem1, osem0, osem1, isem):
    wid = lax.axis_index("s") * NC + lax.axis_index("c")
    base = wid * EPT
    gsems = (gsem0, gsem1)
    osems = (osem0, osem1)

    pltpu.async_copy(snd3_hbm.at[wid], idx_s2, isem).wait()
    pltpu.async_copy(rcv3_hbm.at[wid], idx_r2, isem).wait()

    def issue_gather(i, b):
        pltpu.async_copy(sp_hbm.at[idx_s2.at[i]], rows_s.at[b], gsems[b])
        pltpu.async_copy(rp_hbm.at[idx_r2.at[i]], rows_r.at[b], gsems[b])

    def wait_gather(i, b):
        pltpu.make_async_copy(sp_hbm.at[idx_s2.at[i]], rows_s.at[b], gsems[b]).wait()
        pltpu.make_async_copy(rp_hbm.at[idx_r2.at[i]], rows_r.at[b], gsems[b]).wait()

    def process(i, b, prefetch):
        if prefetch:
            issue_gather(i + 1, 1 - b)
        wait_gather(i, b)

        @pl.when(i >= NBUF)
        def _():
            pltpu.make_async_copy(rows_o.at[b], out_hbm.at[pl.ds(base, KC)],
                                  osems[b]).wait()

        def add_row(j, c2):
            for l in range(D // 16):
                sl = pl.ds(l * 16, 16)
                rows_o[b, j, sl] = rows_s[b, j, sl] + rows_r[b, j, sl]
            return c2

        lax.fori_loop(0, KC, add_row, 0, unroll=8)
        pltpu.async_copy(rows_o.at[b], out_hbm.at[pl.ds(base + i * KC, KC)],
                         osems[b])

    issue_gather(0, 0)

    @pl.loop(0, NCH - 1, step=NBUF)
    def _(g):
        for b in range(NBUF):
            process(g + b, b, True)

    process(NCH - 1, (NCH - 1) % NBUF, False)
    pltpu.make_async_copy(rows_o.at[0], out_hbm.at[pl.ds(base, KC)], osem0).wait()
    pltpu.make_async_copy(rows_o.at[1], out_hbm.at[pl.ds(base, KC)], osem1).wait()


def _edge_body(gs_ref, el_ref, w1_ref, b1_ref, w2_ref, b2_ref, g_ref, bt_ref,
               me_ref, out_ref):
    el = el_ref[...]
    h1 = gs_ref[...] + jnp.dot(el, w1_ref[...], preferred_element_type=jnp.float32)
    h1 = jnp.maximum(h1 + b1_ref[...], 0.0)
    h2 = jnp.dot(h1, w2_ref[...], preferred_element_type=jnp.float32)
    h2 = jnp.maximum(h2 + b2_ref[...], 0.0)
    mu = jnp.mean(h2, axis=-1, keepdims=True)
    var = jnp.mean((h2 - mu) ** 2, axis=-1, keepdims=True)
    ln = (h2 - mu) * lax.rsqrt(var + 1e-5) * g_ref[...] + bt_ref[...]
    me_ref[...] = ln
    out_ref[...] = ln + el


def _edge_mlp(gsum, edge_latents, w1c, b1, w2, b2, g, bt):
    be = 2000
    grid = (E // be,)
    row = lambda i: (i, 0)
    full = lambda i: (0, 0)
    return pl.pallas_call(
        _edge_body,
        grid=grid,
        in_specs=[
            pl.BlockSpec((be, D), row),
            pl.BlockSpec((be, D), row),
            pl.BlockSpec((D, D), full),
            pl.BlockSpec((1, D), full),
            pl.BlockSpec((D, D), full),
            pl.BlockSpec((1, D), full),
            pl.BlockSpec((1, D), full),
            pl.BlockSpec((1, D), full),
        ],
        out_specs=[
            pl.BlockSpec((be, D), row),
            pl.BlockSpec((be, D), row),
        ],
        out_shape=[
            jax.ShapeDtypeStruct((E, D), jnp.float32),
            jax.ShapeDtypeStruct((E, D), jnp.float32),
        ],
    )(gsum, edge_latents, w1c, b1, w2, b2, g, bt)


@functools.partial(
    pl.kernel,
    out_type=jax.ShapeDtypeStruct((NC, NPAD, D), jnp.float32),
    mesh=_MESH,
    scratch_types=[
        pltpu.VMEM((NCHP, KC), jnp.int32),
        pltpu.VMEM((NBUF, KC, D), jnp.float32),
        pltpu.VMEM_SHARED((NPAD, D), jnp.float32),
        pltpu.SemaphoreType.DMA,
        pltpu.SemaphoreType.DMA,
        pltpu.SemaphoreType.DMA,
        pltpu.SemaphoreType.DMA,
        pltpu.SemaphoreType.DMA,
    ],
)
def _scatter_add(me_hbm, rcv3_hbm, zeros_hbm, out_hbm,
                 idx2, rows, acc, rsem0, rsem1, ssem0, ssem1, isem):
    c = lax.axis_index("c")
    s = lax.axis_index("s")
    wid = s * NC + c
    rbase = s * ROWS_PT
    base = wid * EPT
    rsems = (rsem0, rsem1)
    ssems = (ssem0, ssem1)

    zcp = pltpu.async_copy(zeros_hbm.at[pl.ds(rbase, ROWS_PT)],
                           acc.at[pl.ds(rbase, ROWS_PT)], isem)
    pltpu.async_copy(rcv3_hbm.at[wid], idx2, isem)
    zcp.wait()
    pltpu.make_async_copy(rcv3_hbm.at[wid], idx2, isem).wait()
    plsc.subcore_barrier()

    def load_rows(i, b):
        pltpu.async_copy(me_hbm.at[pl.ds(base + i * KC, KC)], rows.at[b],
                         rsems[b])

    def process(i, b, prefetch):
        @pl.when(i >= 1)
        def _():
            pltpu.make_async_copy(rows.at[1 - b], acc.at[idx2.at[i]],
                                  ssems[1 - b]).wait()
        if prefetch:
            load_rows(i + 1, 1 - b)
        pltpu.make_async_copy(me_hbm.at[pl.ds(base, KC)], rows.at[b],
                              rsems[b]).wait()
        pltpu.async_copy(rows.at[b], acc.at[idx2.at[i]], ssems[b], add=True)

    load_rows(0, 0)

    @pl.loop(0, NCH - 1, step=NBUF)
    def _(g):
        for b in range(NBUF):
            process(g + b, b, True)

    process(NCH - 1, (NCH - 1) % NBUF, False)
    pltpu.make_async_copy(rows.at[(NCH - 1) % NBUF], acc.at[idx2.at[0]],
                          ssems[(NCH - 1) % NBUF]).wait()
    plsc.subcore_barrier()
    pltpu.sync_copy(acc.at[pl.ds(rbase, ROWS_PT)],
                    out_hbm.at[c, pl.ds(rbase, ROWS_PT)])


def _node_body(nd_ref, p_ref, w1a_ref, w1b_ref, b1_ref, w2_ref, b2_ref,
               g_ref, bt_ref, out_ref):
    nd = nd_ref[...]
    aggr = p_ref[0] + p_ref[1]
    h1 = (jnp.dot(nd, w1a_ref[...], preferred_element_type=jnp.float32)
          + jnp.dot(aggr, w1b_ref[...], preferred_element_type=jnp.float32))
    h1 = jnp.maximum(h1 + b1_ref[...], 0.0)
    h2 = jnp.dot(h1, w2_ref[...], preferred_element_type=jnp.float32)
    h2 = jnp.maximum(h2 + b2_ref[...], 0.0)
    mu = jnp.mean(h2, axis=-1, keepdims=True)
    var = jnp.mean((h2 - mu) ** 2, axis=-1, keepdims=True)
    ln = (h2 - mu) * lax.rsqrt(var + 1e-5) * g_ref[...] + bt_ref[...]
    out_ref[...] = ln + nd


def _node_mlp(node, partials, w1a, w1b, b1, w2, b2, g, bt):
    bn = 1000
    grid = (N // bn,)
    row = lambda i: (i, 0)
    full = lambda i: (0, 0)
    return pl.pallas_call(
        _node_body,
        grid=grid,
        in_specs=[
            pl.BlockSpec((bn, D), row),
            pl.BlockSpec((NC, bn, D), lambda i: (0, i, 0)),
            pl.BlockSpec((D, D), full),
            pl.BlockSpec((D, D), full),
            pl.BlockSpec((1, D), full),
            pl.BlockSpec((D, D), full),
            pl.BlockSpec((1, D), full),
            pl.BlockSpec((1, D), full),
            pl.BlockSpec((1, D), full),
        ],
        out_specs=pl.BlockSpec((bn, D), row),
        out_shape=jax.ShapeDtypeStruct((N, D), jnp.float32),
    )(node, partials, w1a, w1b, b1, w2, b2, g, bt)


def kernel(node_latents, mesh_edge_latents, senders, receivers,
           eW1, eb1, eW2, eb2, eg, ebt,
           nW1, nb1, nW2, nb2, ng, nbt):
    senders = senders.astype(jnp.int32)
    receivers = receivers.astype(jnp.int32)

    def to3d(x):
        x = x.reshape(NW, NCH, KC)
        return jnp.pad(x, ((0, 0), (0, NCHP - NCH), (0, 0)))

    snd3 = to3d(senders)
    rcv3 = to3d(receivers)

    sp, rp = _project(node_latents, eW1[:D], eW1[D:2 * D])
    gsum = _gather_sum(sp, rp, snd3, rcv3)
    me, new_edges = _edge_mlp(
        gsum, mesh_edge_latents, eW1[2 * D:],
        eb1.reshape(1, D), eW2, eb2.reshape(1, D),
        eg.reshape(1, D), ebt.reshape(1, D))
    zeros = jnp.zeros((NPAD, D), jnp.float32)
    partials = _scatter_add(me, rcv3, zeros)
    new_nodes = _node_mlp(
        node_latents, partials, nW1[:D], nW1[D:],
        nb1.reshape(1, D), nW2, nb2.reshape(1, D),
        ng.reshape(1, D), nbt.reshape(1, D))
    return (new_nodes, new_edges)


# speedup vs baseline: 4.2450x; 1.1343x over previous
import functools

import jax
import jax.numpy as jnp
from jax import lax
from jax.experimental import pallas as pl
from jax.experimental.pallas import tpu as pltpu
from jax.experimental.pallas import tpu_sc as plsc

N = 10000
E = 320000
D = 128

NC = 2
NS = 16
NW = NC * NS

EPT = E // NW
KC = 80
NCH = EPT // KC

NCHP = 128
NPAD = 10240
ROWS_PT = NPAD // NS
NBUF = 2

_MESH = plsc.VectorSubcoreMesh(core_axis_name="c", subcore_axis_name="s")


def _proj_body(node_ref, wa_ref, wb_ref, sp_ref, rp_ref):
    n = node_ref[...]
    sp_ref[...] = jnp.dot(n, wa_ref[...], preferred_element_type=jnp.float32)
    rp_ref[...] = jnp.dot(n, wb_ref[...], preferred_element_type=jnp.float32)


def _project(node, wa, wb):
    bn = 1000
    grid = (N // bn,)
    return pl.pallas_call(
        _proj_body,
        grid=grid,
        in_specs=[
            pl.BlockSpec((bn, D), lambda i: (i, 0)),
            pl.BlockSpec((D, D), lambda i: (0, 0)),
            pl.BlockSpec((D, D), lambda i: (0, 0)),
        ],
        out_specs=[
            pl.BlockSpec((bn, D), lambda i: (i, 0)),
            pl.BlockSpec((bn, D), lambda i: (i, 0)),
        ],
        out_shape=[
            jax.ShapeDtypeStruct((N, D), jnp.float32),
            jax.ShapeDtypeStruct((N, D), jnp.float32),
        ],
    )(node, wa, wb)


GBUF = 3
GMAIN = (NCH // GBUF) * GBUF - GBUF


@functools.partial(
    pl.kernel,
    out_type=jax.ShapeDtypeStruct((E, D), jnp.float32),
    mesh=_MESH,
    scratch_types=[
        pltpu.VMEM((NCHP, KC), jnp.int32),
        pltpu.VMEM((NCHP, KC), jnp.int32),
        pltpu.VMEM((GBUF, KC, D), jnp.float32),
        pltpu.VMEM((GBUF, KC, D), jnp.float32),
        pltpu.VMEM((GBUF, KC, D), jnp.float32),
        [pltpu.SemaphoreType.DMA] * GBUF,
        [pltpu.SemaphoreType.DMA] * GBUF,
        pltpu.SemaphoreType.DMA,
    ],
)
def _gather_sum(sp_hbm, rp_hbm, snd3_hbm, rcv3_hbm, out_hbm,
                idx_s2, idx_r2, rows_s, rows_r, rows_o,
                gsems, osems, isem):
    wid = lax.axis_index("s") * NC + lax.axis_index("c")
    base = wid * EPT

    pltpu.async_copy(snd3_hbm.at[wid], idx_s2, isem).wait()
    pltpu.async_copy(rcv3_hbm.at[wid], idx_r2, isem).wait()

    def issue_gather(i, b):
        pltpu.async_copy(sp_hbm.at[idx_s2.at[i]], rows_s.at[b], gsems[b])
        pltpu.async_copy(rp_hbm.at[idx_r2.at[i]], rows_r.at[b], gsems[b])

    def wait_gather(i, b):
        pltpu.make_async_copy(sp_hbm.at[idx_s2.at[i]], rows_s.at[b], gsems[b]).wait()
        pltpu.make_async_copy(rp_hbm.at[idx_r2.at[i]], rows_r.at[b], gsems[b]).wait()

    def process(i, b, prefetch):
        if prefetch:
            issue_gather(i + (GBUF - 1), (b + GBUF - 1) % GBUF)
        wait_gather(i, b)

        @pl.when(i >= GBUF)
        def _():
            pltpu.make_async_copy(rows_o.at[b], out_hbm.at[pl.ds(base, KC)],
                                  osems[b]).wait()

        def add_row(j, c2):
            for l in range(D // 16):
                sl = pl.ds(l * 16, 16)
                rows_o[b, j, sl] = rows_s[b, j, sl] + rows_r[b, j, sl]
            return c2

        lax.fori_loop(0, KC, add_row, 0, unroll=8)
        pltpu.async_copy(rows_o.at[b], out_hbm.at[pl.ds(base + i * KC, KC)],
                         osems[b])

    for b in range(GBUF - 1):
        issue_gather(b, b)

    @pl.loop(0, GMAIN, step=GBUF)
    def _(g):
        for b in range(GBUF):
            process(g + b, b, True)

    for i in range(GMAIN, NCH):
        process(i, i % GBUF, i + (GBUF - 1) < NCH)

    for b in range(GBUF):
        pltpu.make_async_copy(rows_o.at[b], out_hbm.at[pl.ds(base, KC)],
                              osems[b]).wait()


def _edge_body(gs_ref, el_ref, w1_ref, b1_ref, w2_ref, b2_ref, g_ref, bt_ref,
               me_ref, out_ref):
    el = el_ref[...]
    h1 = gs_ref[...] + jnp.dot(el, w1_ref[...], preferred_element_type=jnp.float32)
    h1 = jnp.maximum(h1 + b1_ref[...], 0.0)
    h2 = jnp.dot(h1, w2_ref[...], preferred_element_type=jnp.float32)
    h2 = jnp.maximum(h2 + b2_ref[...], 0.0)
    mu = jnp.mean(h2, axis=-1, keepdims=True)
    var = jnp.mean((h2 - mu) ** 2, axis=-1, keepdims=True)
    ln = (h2 - mu) * lax.rsqrt(var + 1e-5) * g_ref[...] + bt_ref[...]
    me_ref[...] = ln
    out_ref[...] = ln + el


def _edge_mlp(gsum, edge_latents, w1c, b1, w2, b2, g, bt):
    be = 2000
    grid = (E // be,)
    row = lambda i: (i, 0)
    full = lambda i: (0, 0)
    return pl.pallas_call(
        _edge_body,
        grid=grid,
        in_specs=[
            pl.BlockSpec((be, D), row),
            pl.BlockSpec((be, D), row),
            pl.BlockSpec((D, D), full),
            pl.BlockSpec((1, D), full),
            pl.BlockSpec((D, D), full),
            pl.BlockSpec((1, D), full),
            pl.BlockSpec((1, D), full),
            pl.BlockSpec((1, D), full),
        ],
        out_specs=[
            pl.BlockSpec((be, D), row),
            pl.BlockSpec((be, D), row),
        ],
        out_shape=[
            jax.ShapeDtypeStruct((E, D), jnp.float32),
            jax.ShapeDtypeStruct((E, D), jnp.float32),
        ],
    )(gsum, edge_latents, w1c, b1, w2, b2, g, bt)


@functools.partial(
    pl.kernel,
    out_type=jax.ShapeDtypeStruct((NC, NPAD, D), jnp.float32),
    mesh=_MESH,
    scratch_types=[
        pltpu.VMEM((NCHP, KC), jnp.int32),
        pltpu.VMEM((NBUF, KC, D), jnp.float32),
        pltpu.VMEM_SHARED((NPAD, D), jnp.float32),
        pltpu.SemaphoreType.DMA,
        pltpu.SemaphoreType.DMA,
        pltpu.SemaphoreType.DMA,
        pltpu.SemaphoreType.DMA,
        pltpu.SemaphoreType.DMA,
    ],
)
def _scatter_add(me_hbm, rcv3_hbm, zeros_hbm, out_hbm,
                 idx2, rows, acc, rsem0, rsem1, ssem0, ssem1, isem):
    c = lax.axis_index("c")
    s = lax.axis_index("s")
    wid = s * NC + c
    rbase = s * ROWS_PT
    base = wid * EPT
    rsems = (rsem0, rsem1)
    ssems = (ssem0, ssem1)

    zcp = pltpu.async_copy(zeros_hbm.at[pl.ds(rbase, ROWS_PT)],
                           acc.at[pl.ds(rbase, ROWS_PT)], isem)
    pltpu.async_copy(rcv3_hbm.at[wid], idx2, isem)
    zcp.wait()
    pltpu.make_async_copy(rcv3_hbm.at[wid], idx2, isem).wait()
    plsc.subcore_barrier()

    def load_rows(i, b):
        pltpu.async_copy(me_hbm.at[pl.ds(base + i * KC, KC)], rows.at[b],
                         rsems[b])

    def process(i, b, prefetch):
        @pl.when(i >= 1)
        def _():
            pltpu.make_async_copy(rows.at[1 - b], acc.at[idx2.at[i]],
                                  ssems[1 - b]).wait()
        if prefetch:
            load_rows(i + 1, 1 - b)
        pltpu.make_async_copy(me_hbm.at[pl.ds(base, KC)], rows.at[b],
                              rsems[b]).wait()
        pltpu.async_copy(rows.at[b], acc.at[idx2.at[i]], ssems[b], add=True)

    load_rows(0, 0)

    @pl.loop(0, NCH - 1, step=NBUF)
    def _(g):
        for b in range(NBUF):
            process(g + b, b, True)

    process(NCH - 1, (NCH - 1) % NBUF, False)
    pltpu.make_async_copy(rows.at[(NCH - 1) % NBUF], acc.at[idx2.at[0]],
                          ssems[(NCH - 1) % NBUF]).wait()
    plsc.subcore_barrier()
    pltpu.sync_copy(acc.at[pl.ds(rbase, ROWS_PT)],
                    out_hbm.at[c, pl.ds(rbase, ROWS_PT)])


def _node_body(nd_ref, p_ref, w1a_ref, w1b_ref, b1_ref, w2_ref, b2_ref,
               g_ref, bt_ref, out_ref):
    nd = nd_ref[...]
    aggr = p_ref[0] + p_ref[1]
    h1 = (jnp.dot(nd, w1a_ref[...], preferred_element_type=jnp.float32)
          + jnp.dot(aggr, w1b_ref[...], preferred_element_type=jnp.float32))
    h1 = jnp.maximum(h1 + b1_ref[...], 0.0)
    h2 = jnp.dot(h1, w2_ref[...], preferred_element_type=jnp.float32)
    h2 = jnp.maximum(h2 + b2_ref[...], 0.0)
    mu = jnp.mean(h2, axis=-1, keepdims=True)
    var = jnp.mean((h2 - mu) ** 2, axis=-1, keepdims=True)
    ln = (h2 - mu) * lax.rsqrt(var + 1e-5) * g_ref[...] + bt_ref[...]
    out_ref[...] = ln + nd


def _node_mlp(node, partials, w1a, w1b, b1, w2, b2, g, bt):
    bn = 1000
    grid = (N // bn,)
    row = lambda i: (i, 0)
    full = lambda i: (0, 0)
    return pl.pallas_call(
        _node_body,
        grid=grid,
        in_specs=[
            pl.BlockSpec((bn, D), row),
            pl.BlockSpec((NC, bn, D), lambda i: (0, i, 0)),
            pl.BlockSpec((D, D), full),
            pl.BlockSpec((D, D), full),
            pl.BlockSpec((1, D), full),
            pl.BlockSpec((D, D), full),
            pl.BlockSpec((1, D), full),
            pl.BlockSpec((1, D), full),
            pl.BlockSpec((1, D), full),
        ],
        out_specs=pl.BlockSpec((bn, D), row),
        out_shape=jax.ShapeDtypeStruct((N, D), jnp.float32),
    )(node, partials, w1a, w1b, b1, w2, b2, g, bt)


def kernel(node_latents, mesh_edge_latents, senders, receivers,
           eW1, eb1, eW2, eb2, eg, ebt,
           nW1, nb1, nW2, nb2, ng, nbt):
    senders = senders.astype(jnp.int32)
    receivers = receivers.astype(jnp.int32)

    def to3d(x):
        x = x.reshape(NW, NCH, KC)
        return jnp.pad(x, ((0, 0), (0, NCHP - NCH), (0, 0)))

    snd3 = to3d(senders)
    rcv3 = to3d(receivers)

    sp, rp = _project(node_latents, eW1[:D], eW1[D:2 * D])
    gsum = _gather_sum(sp, rp, snd3, rcv3)
    me, new_edges = _edge_mlp(
        gsum, mesh_edge_latents, eW1[2 * D:],
        eb1.reshape(1, D), eW2, eb2.reshape(1, D),
        eg.reshape(1, D), ebt.reshape(1, D))
    zeros = jnp.zeros((NPAD, D), jnp.float32)
    partials = _scatter_add(me, rcv3, zeros)
    new_nodes = _node_mlp(
        node_latents, partials, nW1[:D], nW1[D:],
        nb1.reshape(1, D), nW2, nb2.reshape(1, D),
        ng.reshape(1, D), nbt.reshape(1, D))
    return (new_nodes, new_edges)


# speedup vs baseline: 4.9713x; 1.1711x over previous
import functools

import jax
import jax.numpy as jnp
from jax import lax
from jax.experimental import pallas as pl
from jax.experimental.pallas import tpu as pltpu
from jax.experimental.pallas import tpu_sc as plsc

N = 10000
E = 320000
D = 128

NC = 2
NS = 16
NW = NC * NS

S = 5
ES = E // S
KC = 80
EPT = ES // NW
NCH = EPT // KC
NCHP = 32

NPAD = 10240
ROWS_PT = NPAD // NS
NBUF = 2

_MESH = plsc.VectorSubcoreMesh(core_axis_name="c", subcore_axis_name="s")


def _proj_body(node_ref, wa_ref, wb_ref, sp_ref, rp_ref):
    n = node_ref[...]
    sp_ref[...] = jnp.dot(n, wa_ref[...], preferred_element_type=jnp.float32)
    rp_ref[...] = jnp.dot(n, wb_ref[...], preferred_element_type=jnp.float32)


def _project(node, wa, wb):
    bn = 1000
    grid = (N // bn,)
    return pl.pallas_call(
        _proj_body,
        grid=grid,
        in_specs=[
            pl.BlockSpec((bn, D), lambda i: (i, 0)),
            pl.BlockSpec((D, D), lambda i: (0, 0)),
            pl.BlockSpec((D, D), lambda i: (0, 0)),
        ],
        out_specs=[
            pl.BlockSpec((bn, D), lambda i: (i, 0)),
            pl.BlockSpec((bn, D), lambda i: (i, 0)),
        ],
        out_shape=[
            jax.ShapeDtypeStruct((N, D), jnp.float32),
            jax.ShapeDtypeStruct((N, D), jnp.float32),
        ],
    )(node, wa, wb)


GBUF = 3
GMAIN = (NCH // GBUF) * GBUF - GBUF


@functools.partial(
    pl.kernel,
    out_type=jax.ShapeDtypeStruct((ES, D), jnp.float32),
    mesh=_MESH,
    scratch_types=[
        pltpu.VMEM((NCHP, KC), jnp.int32),
        pltpu.VMEM((NCHP, KC), jnp.int32),
        pltpu.VMEM((GBUF, KC, D), jnp.float32),
        pltpu.VMEM((GBUF, KC, D), jnp.float32),
        pltpu.VMEM((GBUF, KC, D), jnp.float32),
        [pltpu.SemaphoreType.DMA] * GBUF,
        [pltpu.SemaphoreType.DMA] * GBUF,
        pltpu.SemaphoreType.DMA,
    ],
)
def _gather_sum(sp_hbm, rp_hbm, snd3_hbm, rcv3_hbm, out_hbm,
                idx_s2, idx_r2, rows_s, rows_r, rows_o,
                gsems, osems, isem):
    wid = lax.axis_index("s") * NC + lax.axis_index("c")
    base = wid * EPT

    pltpu.async_copy(snd3_hbm.at[wid], idx_s2, isem).wait()
    pltpu.async_copy(rcv3_hbm.at[wid], idx_r2, isem).wait()

    def issue_gather(i, b):
        pltpu.async_copy(sp_hbm.at[idx_s2.at[i]], rows_s.at[b], gsems[b])
        pltpu.async_copy(rp_hbm.at[idx_r2.at[i]], rows_r.at[b], gsems[b])

    def wait_gather(i, b):
        pltpu.make_async_copy(sp_hbm.at[idx_s2.at[i]], rows_s.at[b], gsems[b]).wait()
        pltpu.make_async_copy(rp_hbm.at[idx_r2.at[i]], rows_r.at[b], gsems[b]).wait()

    def process(i, b, prefetch):
        if prefetch:
            issue_gather(i + (GBUF - 1), (b + GBUF - 1) % GBUF)
        wait_gather(i, b)

        @pl.when(i >= GBUF)
        def _():
            pltpu.make_async_copy(rows_o.at[b], out_hbm.at[pl.ds(base, KC)],
                                  osems[b]).wait()

        def add_row(j, c2):
            for l in range(D // 16):
                sl = pl.ds(l * 16, 16)
                rows_o[b, j, sl] = rows_s[b, j, sl] + rows_r[b, j, sl]
            return c2

        lax.fori_loop(0, KC, add_row, 0, unroll=8)
        pltpu.async_copy(rows_o.at[b], out_hbm.at[pl.ds(base + i * KC, KC)],
                         osems[b])

    for b in range(GBUF - 1):
        issue_gather(b, b)

    @pl.loop(0, GMAIN, step=GBUF)
    def _(g):
        for b in range(GBUF):
            process(g + b, b, True)

    for i in range(GMAIN, NCH):
        process(i, i % GBUF, i + (GBUF - 1) < NCH)

    for b in range(GBUF):
        pltpu.make_async_copy(rows_o.at[b], out_hbm.at[pl.ds(base, KC)],
                              osems[b]).wait()


def _edge_body(gs_ref, el_ref, w1_ref, b1_ref, w2_ref, b2_ref, g_ref, bt_ref,
               alias_ref, me_ref, out_ref):
    del alias_ref
    el = el_ref[...]
    h1 = gs_ref[...] + jnp.dot(el, w1_ref[...], preferred_element_type=jnp.float32)
    h1 = jnp.maximum(h1 + b1_ref[...], 0.0)
    h2 = jnp.dot(h1, w2_ref[...], preferred_element_type=jnp.float32)
    h2 = jnp.maximum(h2 + b2_ref[...], 0.0)
    mu = jnp.mean(h2, axis=-1, keepdims=True)
    var = jnp.mean((h2 - mu) ** 2, axis=-1, keepdims=True)
    ln = (h2 - mu) * lax.rsqrt(var + 1e-5) * g_ref[...] + bt_ref[...]
    me_ref[...] = ln
    out_ref[...] = ln + el


_BE = 2000
_NBLK = ES // _BE


def _edge_mlp(sl, gsum_s, edge_latents, w1c, b1, w2, b2, g, bt, out_carry):
    row = lambda i: (i, 0)
    off_row = lambda i, sl=sl: (sl * _NBLK + i, 0)
    full = lambda i: (0, 0)
    args = [gsum_s, edge_latents, w1c, b1, w2, b2, g, bt]
    in_specs = [
        pl.BlockSpec((_BE, D), row),
        pl.BlockSpec((_BE, D), off_row),
        pl.BlockSpec((D, D), full),
        pl.BlockSpec((1, D), full),
        pl.BlockSpec((D, D), full),
        pl.BlockSpec((1, D), full),
        pl.BlockSpec((1, D), full),
        pl.BlockSpec((1, D), full),
        pl.BlockSpec(memory_space=pl.ANY),
    ]
    if out_carry is None:
        out_carry = jnp.zeros((8, D), jnp.float32)
        in_specs[-1] = pl.BlockSpec(memory_space=pl.ANY)
        io_alias = {}
    else:
        io_alias = {8: 1}
    return pl.pallas_call(
        _edge_body,
        grid=(_NBLK,),
        in_specs=in_specs,
        out_specs=[
            pl.BlockSpec((_BE, D), row),
            pl.BlockSpec((_BE, D), off_row),
        ],
        out_shape=[
            jax.ShapeDtypeStruct((ES, D), jnp.float32),
            jax.ShapeDtypeStruct((E, D), jnp.float32),
        ],
        input_output_aliases=io_alias,
    )(*args, out_carry)


@functools.partial(
    pl.kernel,
    out_type=jax.ShapeDtypeStruct((NC, NPAD, D), jnp.float32),
    mesh=_MESH,
    scratch_types=[
        pltpu.VMEM((NCHP, KC), jnp.int32),
        pltpu.VMEM((NBUF, KC, D), jnp.float32),
        pltpu.VMEM_SHARED((NPAD, D), jnp.float32),
        [pltpu.SemaphoreType.DMA] * NBUF,
        [pltpu.SemaphoreType.DMA] * NBUF,
        pltpu.SemaphoreType.DMA,
    ],
)
def _scatter_add(me0, me1, me2, me3, me4, rcv4_hbm, zeros_hbm, out_hbm,
                 idx2, rows, acc, rsems, ssems, isem):
    c = lax.axis_index("c")
    s = lax.axis_index("s")
    wid = s * NC + c
    rbase = s * ROWS_PT
    base = wid * EPT
    mes = (me0, me1, me2, me3, me4)

    zcp = pltpu.async_copy(zeros_hbm.at[pl.ds(rbase, ROWS_PT)],
                           acc.at[pl.ds(rbase, ROWS_PT)], isem)
    zcp.wait()
    plsc.subcore_barrier()

    for sl in range(S):
        me_hbm = mes[sl]
        pltpu.async_copy(rcv4_hbm.at[sl, wid], idx2, isem).wait()

        def load_rows(i, b, me_hbm=me_hbm):
            pltpu.async_copy(me_hbm.at[pl.ds(base + i * KC, KC)], rows.at[b],
                             rsems[b])

        def process(i, b, prefetch, me_hbm=me_hbm, load_rows=load_rows):
            @pl.when(i >= 1)
            def _():
                pltpu.make_async_copy(rows.at[1 - b], acc.at[idx2.at[i]],
                                      ssems[1 - b]).wait()
            if prefetch:
                load_rows(i + 1, 1 - b)
            pltpu.make_async_copy(me_hbm.at[pl.ds(base, KC)], rows.at[b],
                                  rsems[b]).wait()
            pltpu.async_copy(rows.at[b], acc.at[idx2.at[i]], ssems[b], add=True)

        load_rows(0, 0)

        @pl.loop(0, NCH - 1, step=NBUF)
        def _(g):
            for b in range(NBUF):
                process(g + b, b, True)

        process(NCH - 1, (NCH - 1) % NBUF, False)
        pltpu.make_async_copy(rows.at[(NCH - 1) % NBUF], acc.at[idx2.at[0]],
                              ssems[(NCH - 1) % NBUF]).wait()

    plsc.subcore_barrier()
    pltpu.sync_copy(acc.at[pl.ds(rbase, ROWS_PT)],
                    out_hbm.at[c, pl.ds(rbase, ROWS_PT)])


def _node_body(nd_ref, p_ref, w1a_ref, w1b_ref, b1_ref, w2_ref, b2_ref,
               g_ref, bt_ref, out_ref):
    nd = nd_ref[...]
    aggr = p_ref[0] + p_ref[1]
    h1 = (jnp.dot(nd, w1a_ref[...], preferred_element_type=jnp.float32)
          + jnp.dot(aggr, w1b_ref[...], preferred_element_type=jnp.float32))
    h1 = jnp.maximum(h1 + b1_ref[...], 0.0)
    h2 = jnp.dot(h1, w2_ref[...], preferred_element_type=jnp.float32)
    h2 = jnp.maximum(h2 + b2_ref[...], 0.0)
    mu = jnp.mean(h2, axis=-1, keepdims=True)
    var = jnp.mean((h2 - mu) ** 2, axis=-1, keepdims=True)
    ln = (h2 - mu) * lax.rsqrt(var + 1e-5) * g_ref[...] + bt_ref[...]
    out_ref[...] = ln + nd


def _node_mlp(node, partials, w1a, w1b, b1, w2, b2, g, bt):
    bn = 1000
    grid = (N // bn,)
    row = lambda i: (i, 0)
    full = lambda i: (0, 0)
    return pl.pallas_call(
        _node_body,
        grid=grid,
        in_specs=[
            pl.BlockSpec((bn, D), row),
            pl.BlockSpec((NC, bn, D), lambda i: (0, i, 0)),
            pl.BlockSpec((D, D), full),
            pl.BlockSpec((D, D), full),
            pl.BlockSpec((1, D), full),
            pl.BlockSpec((D, D), full),
            pl.BlockSpec((1, D), full),
            pl.BlockSpec((1, D), full),
            pl.BlockSpec((1, D), full),
        ],
        out_specs=pl.BlockSpec((bn, D), row),
        out_shape=jax.ShapeDtypeStruct((N, D), jnp.float32),
    )(node, partials, w1a, w1b, b1, w2, b2, g, bt)


def kernel(node_latents, mesh_edge_latents, senders, receivers,
           eW1, eb1, eW2, eb2, eg, ebt,
           nW1, nb1, nW2, nb2, ng, nbt):
    senders = senders.astype(jnp.int32)
    receivers = receivers.astype(jnp.int32)

    def to4d(x):
        x = x.reshape(S, NW, NCH, KC)
        return jnp.pad(x, ((0, 0), (0, 0), (0, NCHP - NCH), (0, 0)))

    snd4 = to4d(senders)
    rcv4 = to4d(receivers)

    sp, rp = _project(node_latents, eW1[:D], eW1[D:2 * D])

    eb1r = eb1.reshape(1, D)
    eb2r = eb2.reshape(1, D)
    egr = eg.reshape(1, D)
    ebtr = ebt.reshape(1, D)
    w1c = eW1[2 * D:]

    mes = []
    out_carry = None
    for sl in range(S):
        gsum_s = _gather_sum(sp, rp, snd4[sl], rcv4[sl])
        me_s, out_carry = _edge_mlp(sl, gsum_s, mesh_edge_latents, w1c,
                                    eb1r, eW2, eb2r, egr, ebtr, out_carry)
        mes.append(me_s)
    new_edges = out_carry

    zeros = jnp.zeros((NPAD, D), jnp.float32)
    partials = _scatter_add(*mes, rcv4, zeros)
    new_nodes = _node_mlp(
        node_latents, partials, nW1[:D], nW1[D:],
        nb1.reshape(1, D), nW2, nb2.reshape(1, D),
        ng.reshape(1, D), nbt.reshape(1, D))
    return (new_nodes, new_edges)


# speedup vs baseline: 5.1016x; 1.0262x over previous
import functools

import jax
import jax.numpy as jnp
from jax import lax
from jax.experimental import pallas as pl
from jax.experimental.pallas import tpu as pltpu
from jax.experimental.pallas import tpu_sc as plsc

N = 10000
E = 320000
D = 128

NC = 2
NS = 16
NW = NC * NS

S = 5
ES = E // S
KC = 80
EPT = ES // NW
NCH = EPT // KC
NCHP = 32

NPAD = 10240
ROWS_PT = NPAD // NS
NBUF = 2

_MESH = plsc.VectorSubcoreMesh(core_axis_name="c", subcore_axis_name="s")


def _proj_body(node_ref, wa_ref, wb_ref, sp_ref, rp_ref):
    n = node_ref[...]
    sp_ref[...] = jnp.dot(n, wa_ref[...], preferred_element_type=jnp.float32)
    rp_ref[...] = jnp.dot(n, wb_ref[...], preferred_element_type=jnp.float32)


def _project(node, wa, wb):
    bn = 1000
    grid = (N // bn,)
    return pl.pallas_call(
        _proj_body,
        grid=grid,
        in_specs=[
            pl.BlockSpec((bn, D), lambda i: (i, 0)),
            pl.BlockSpec((D, D), lambda i: (0, 0)),
            pl.BlockSpec((D, D), lambda i: (0, 0)),
        ],
        out_specs=[
            pl.BlockSpec((bn, D), lambda i: (i, 0)),
            pl.BlockSpec((bn, D), lambda i: (i, 0)),
        ],
        out_shape=[
            jax.ShapeDtypeStruct((N, D), jnp.float32),
            jax.ShapeDtypeStruct((N, D), jnp.float32),
        ],
    )(node, wa, wb)


GBUF = 3
GMAIN = (NCH // GBUF) * GBUF - GBUF


@functools.partial(
    pl.kernel,
    out_type=jax.ShapeDtypeStruct((ES, D), jnp.float32),
    mesh=_MESH,
    scratch_types=[
        pltpu.VMEM((NCHP, KC), jnp.int32),
        pltpu.VMEM((NCHP, KC), jnp.int32),
        pltpu.VMEM((GBUF, KC, D), jnp.float32),
        pltpu.VMEM((GBUF, KC, D), jnp.float32),
        pltpu.VMEM((GBUF, KC, D), jnp.float32),
        [pltpu.SemaphoreType.DMA] * GBUF,
        [pltpu.SemaphoreType.DMA] * GBUF,
        pltpu.SemaphoreType.DMA,
    ],
)
def _gather_sum(sp_hbm, rp_hbm, snd3_hbm, rcv3_hbm, out_hbm,
                idx_s2, idx_r2, rows_s, rows_r, rows_o,
                gsems, osems, isem):
    wid = lax.axis_index("s") * NC + lax.axis_index("c")
    base = wid * EPT

    pltpu.async_copy(snd3_hbm.at[wid], idx_s2, isem).wait()
    pltpu.async_copy(rcv3_hbm.at[wid], idx_r2, isem).wait()

    def issue_gather(i, b):
        pltpu.async_copy(sp_hbm.at[idx_s2.at[i]], rows_s.at[b], gsems[b])
        pltpu.async_copy(rp_hbm.at[idx_r2.at[i]], rows_r.at[b], gsems[b])

    def wait_gather(i, b):
        pltpu.make_async_copy(sp_hbm.at[idx_s2.at[i]], rows_s.at[b], gsems[b]).wait()
        pltpu.make_async_copy(rp_hbm.at[idx_r2.at[i]], rows_r.at[b], gsems[b]).wait()

    def process(i, b, prefetch):
        if prefetch:
            issue_gather(i + (GBUF - 1), (b + GBUF - 1) % GBUF)
        wait_gather(i, b)

        @pl.when(i >= GBUF)
        def _():
            pltpu.make_async_copy(rows_o.at[b], out_hbm.at[pl.ds(base, KC)],
                                  osems[b]).wait()

        def add_row(j, c2):
            for l in range(D // 16):
                sl = pl.ds(l * 16, 16)
                rows_o[b, j, sl] = rows_s[b, j, sl] + rows_r[b, j, sl]
            return c2

        lax.fori_loop(0, KC, add_row, 0, unroll=8)
        pltpu.async_copy(rows_o.at[b], out_hbm.at[pl.ds(base + i * KC, KC)],
                         osems[b])

    for b in range(GBUF - 1):
        issue_gather(b, b)

    @pl.loop(0, GMAIN, step=GBUF)
    def _(g):
        for b in range(GBUF):
            process(g + b, b, True)

    for i in range(GMAIN, NCH):
        process(i, i % GBUF, i + (GBUF - 1) < NCH)

    for b in range(GBUF):
        pltpu.make_async_copy(rows_o.at[b], out_hbm.at[pl.ds(base, KC)],
                              osems[b]).wait()


def _edge_body(gs_ref, el_ref, w1_ref, b1_ref, w2_ref, b2_ref, g_ref, bt_ref,
               alias_ref, me_ref, out_ref):
    del alias_ref
    el = el_ref[...]
    h1 = gs_ref[...] + jnp.dot(el, w1_ref[...], preferred_element_type=jnp.float32)
    h1 = jnp.maximum(h1 + b1_ref[...], 0.0)
    h2 = jnp.dot(h1, w2_ref[...], preferred_element_type=jnp.float32)
    h2 = jnp.maximum(h2 + b2_ref[...], 0.0)
    mu = jnp.mean(h2, axis=-1, keepdims=True)
    var = jnp.mean((h2 - mu) ** 2, axis=-1, keepdims=True)
    ln = (h2 - mu) * lax.rsqrt(var + 1e-5) * g_ref[...] + bt_ref[...]
    me_ref[...] = ln
    out_ref[...] = ln + el


_BE = 2000
_NBLK = ES // _BE


def _edge_mlp(sl, gsum_s, edge_latents, w1c, b1, w2, b2, g, bt, out_carry):
    row = lambda i: (i, 0)
    off_row = lambda i, sl=sl: (sl * _NBLK + i, 0)
    full = lambda i: (0, 0)
    args = [gsum_s, edge_latents, w1c, b1, w2, b2, g, bt]
    in_specs = [
        pl.BlockSpec((_BE, D), row),
        pl.BlockSpec((_BE, D), off_row),
        pl.BlockSpec((D, D), full),
        pl.BlockSpec((1, D), full),
        pl.BlockSpec((D, D), full),
        pl.BlockSpec((1, D), full),
        pl.BlockSpec((1, D), full),
        pl.BlockSpec((1, D), full),
        pl.BlockSpec(memory_space=pl.ANY),
    ]
    if out_carry is None:
        out_carry = jnp.zeros((8, D), jnp.float32)
        in_specs[-1] = pl.BlockSpec(memory_space=pl.ANY)
        io_alias = {}
    else:
        io_alias = {8: 1}
    return pl.pallas_call(
        _edge_body,
        grid=(_NBLK,),
        in_specs=in_specs,
        out_specs=[
            pl.BlockSpec((_BE, D), row),
            pl.BlockSpec((_BE, D), off_row),
        ],
        out_shape=[
            jax.ShapeDtypeStruct((ES, D), jnp.float32),
            jax.ShapeDtypeStruct((E, D), jnp.float32),
        ],
        input_output_aliases=io_alias,
    )(*args, out_carry)


def _make_scatter(slices):
    n_mes = len(slices)

    @functools.partial(
        pl.kernel,
        out_type=jax.ShapeDtypeStruct((NC, NPAD, D), jnp.float32),
        mesh=_MESH,
        scratch_types=[
            pltpu.VMEM((NCHP, KC), jnp.int32),
            pltpu.VMEM((NBUF, KC, D), jnp.float32),
            pltpu.VMEM_SHARED((NPAD, D), jnp.float32),
            [pltpu.SemaphoreType.DMA] * NBUF,
            [pltpu.SemaphoreType.DMA] * NBUF,
            pltpu.SemaphoreType.DMA,
        ],
    )
    def _scatter(*refs):
        mes = refs[:n_mes]
        rcv4_hbm, zeros_hbm, out_hbm, idx2, rows, acc, rsems, ssems, isem = refs[n_mes:]
        _scatter_body(slices, mes, rcv4_hbm, zeros_hbm, out_hbm,
                      idx2, rows, acc, rsems, ssems, isem)

    return _scatter


def _scatter_body(slices, mes, rcv4_hbm, zeros_hbm, out_hbm,
                  idx2, rows, acc, rsems, ssems, isem):
    c = lax.axis_index("c")
    s = lax.axis_index("s")
    wid = s * NC + c
    rbase = s * ROWS_PT
    base = wid * EPT

    zcp = pltpu.async_copy(zeros_hbm.at[pl.ds(rbase, ROWS_PT)],
                           acc.at[pl.ds(rbase, ROWS_PT)], isem)
    zcp.wait()
    plsc.subcore_barrier()

    for k, sl in enumerate(slices):
        me_hbm = mes[k]
        pltpu.async_copy(rcv4_hbm.at[sl, wid], idx2, isem).wait()

        def load_rows(i, b, me_hbm=me_hbm):
            pltpu.async_copy(me_hbm.at[pl.ds(base + i * KC, KC)], rows.at[b],
                             rsems[b])

        def process(i, b, prefetch, me_hbm=me_hbm, load_rows=load_rows):
            @pl.when(i >= 1)
            def _():
                pltpu.make_async_copy(rows.at[1 - b], acc.at[idx2.at[i]],
                                      ssems[1 - b]).wait()
            if prefetch:
                load_rows(i + 1, 1 - b)
            pltpu.make_async_copy(me_hbm.at[pl.ds(base, KC)], rows.at[b],
                                  rsems[b]).wait()
            pltpu.async_copy(rows.at[b], acc.at[idx2.at[i]], ssems[b], add=True)

        load_rows(0, 0)

        @pl.loop(0, NCH - 1, step=NBUF)
        def _(g):
            for b in range(NBUF):
                process(g + b, b, True)

        process(NCH - 1, (NCH - 1) % NBUF, False)
        pltpu.make_async_copy(rows.at[(NCH - 1) % NBUF], acc.at[idx2.at[0]],
                              ssems[(NCH - 1) % NBUF]).wait()

    plsc.subcore_barrier()
    pltpu.sync_copy(acc.at[pl.ds(rbase, ROWS_PT)],
                    out_hbm.at[c, pl.ds(rbase, ROWS_PT)])


_scatter_a = _make_scatter((0, 1, 2))
_scatter_b = _make_scatter((3, 4))


def _node_body(nd_ref, pa_ref, pb_ref, w1a_ref, w1b_ref, b1_ref, w2_ref, b2_ref,
               g_ref, bt_ref, out_ref):
    nd = nd_ref[...]
    aggr = (pa_ref[0] + pa_ref[1]) + (pb_ref[0] + pb_ref[1])
    h1 = (jnp.dot(nd, w1a_ref[...], preferred_element_type=jnp.float32)
          + jnp.dot(aggr, w1b_ref[...], preferred_element_type=jnp.float32))
    h1 = jnp.maximum(h1 + b1_ref[...], 0.0)
    h2 = jnp.dot(h1, w2_ref[...], preferred_element_type=jnp.float32)
    h2 = jnp.maximum(h2 + b2_ref[...], 0.0)
    mu = jnp.mean(h2, axis=-1, keepdims=True)
    var = jnp.mean((h2 - mu) ** 2, axis=-1, keepdims=True)
    ln = (h2 - mu) * lax.rsqrt(var + 1e-5) * g_ref[...] + bt_ref[...]
    out_ref[...] = ln + nd


def _node_mlp(node, pa, pb, w1a, w1b, b1, w2, b2, g, bt):
    bn = 1000
    grid = (N // bn,)
    row = lambda i: (i, 0)
    full = lambda i: (0, 0)
    return pl.pallas_call(
        _node_body,
        grid=grid,
        in_specs=[
            pl.BlockSpec((bn, D), row),
            pl.BlockSpec((NC, bn, D), lambda i: (0, i, 0)),
            pl.BlockSpec((NC, bn, D), lambda i: (0, i, 0)),
            pl.BlockSpec((D, D), full),
            pl.BlockSpec((D, D), full),
            pl.BlockSpec((1, D), full),
            pl.BlockSpec((D, D), full),
            pl.BlockSpec((1, D), full),
            pl.BlockSpec((1, D), full),
            pl.BlockSpec((1, D), full),
        ],
        out_specs=pl.BlockSpec((bn, D), row),
        out_shape=jax.ShapeDtypeStruct((N, D), jnp.float32),
    )(node, pa, pb, w1a, w1b, b1, w2, b2, g, bt)


def kernel(node_latents, mesh_edge_latents, senders, receivers,
           eW1, eb1, eW2, eb2, eg, ebt,
           nW1, nb1, nW2, nb2, ng, nbt):
    senders = senders.astype(jnp.int32)
    receivers = receivers.astype(jnp.int32)

    def to4d(x):
        x = x.reshape(S, NW, NCH, KC)
        return jnp.pad(x, ((0, 0), (0, 0), (0, NCHP - NCH), (0, 0)))

    snd4 = to4d(senders)
    rcv4 = to4d(receivers)

    sp, rp = _project(node_latents, eW1[:D], eW1[D:2 * D])

    eb1r = eb1.reshape(1, D)
    eb2r = eb2.reshape(1, D)
    egr = eg.reshape(1, D)
    ebtr = ebt.reshape(1, D)
    w1c = eW1[2 * D:]

    mes = []
    out_carry = None
    for sl in range(S):
        gsum_s = _gather_sum(sp, rp, snd4[sl], rcv4[sl])
        me_s, out_carry = _edge_mlp(sl, gsum_s, mesh_edge_latents, w1c,
                                    eb1r, eW2, eb2r, egr, ebtr, out_carry)
        mes.append(me_s)
    new_edges = out_carry

    zeros = jnp.zeros((NPAD, D), jnp.float32)
    pa = _scatter_a(mes[0], mes[1], mes[2], rcv4, zeros)
    pb = _scatter_b(mes[3], mes[4], rcv4, zeros)
    new_nodes = _node_mlp(
        node_latents, pa, pb, nW1[:D], nW1[D:],
        nb1.reshape(1, D), nW2, nb2.reshape(1, D),
        ng.reshape(1, D), nbt.reshape(1, D))
    return (new_nodes, new_edges)


# speedup vs baseline: 5.1189x; 1.0034x over previous
import functools

import jax
import jax.numpy as jnp
from jax import lax
from jax.experimental import pallas as pl
from jax.experimental.pallas import tpu as pltpu
from jax.experimental.pallas import tpu_sc as plsc

N = 10000
E = 320000
D = 128

NC = 2
NS = 16
NW = NC * NS

S = 5
ES = E // S
KC = 80
EPT = ES // NW
NCH = EPT // KC
NCHP = 32

NPAD = 10240
ROWS_PT = NPAD // NS
NBUF = 2

_MESH = plsc.VectorSubcoreMesh(core_axis_name="c", subcore_axis_name="s")


def _proj_body(node_ref, wa_ref, wb_ref, sp_ref, rp_ref):
    n = node_ref[...]
    sp_ref[...] = jnp.dot(n, wa_ref[...], preferred_element_type=jnp.float32)
    rp_ref[...] = jnp.dot(n, wb_ref[...], preferred_element_type=jnp.float32)


def _project(node, wa, wb):
    bn = 1000
    grid = (N // bn,)
    return pl.pallas_call(
        _proj_body,
        grid=grid,
        in_specs=[
            pl.BlockSpec((bn, D), lambda i: (i, 0)),
            pl.BlockSpec((D, D), lambda i: (0, 0)),
            pl.BlockSpec((D, D), lambda i: (0, 0)),
        ],
        out_specs=[
            pl.BlockSpec((bn, D), lambda i: (i, 0)),
            pl.BlockSpec((bn, D), lambda i: (i, 0)),
        ],
        out_shape=[
            jax.ShapeDtypeStruct((N, D), jnp.float32),
            jax.ShapeDtypeStruct((N, D), jnp.float32),
        ],
    )(node, wa, wb)


GBUF = 4
GMAIN = (NCH // GBUF) * GBUF - GBUF


@functools.partial(
    pl.kernel,
    out_type=jax.ShapeDtypeStruct((ES, D), jnp.float32),
    mesh=_MESH,
    scratch_types=[
        pltpu.VMEM((NCHP, KC), jnp.int32),
        pltpu.VMEM((NCHP, KC), jnp.int32),
        pltpu.VMEM((GBUF, KC, D), jnp.float32),
        pltpu.VMEM((GBUF, KC, D), jnp.float32),
        pltpu.VMEM((GBUF, KC, D), jnp.float32),
        [pltpu.SemaphoreType.DMA] * GBUF,
        [pltpu.SemaphoreType.DMA] * GBUF,
        pltpu.SemaphoreType.DMA,
    ],
)
def _gather_sum(sp_hbm, rp_hbm, snd3_hbm, rcv3_hbm, out_hbm,
                idx_s2, idx_r2, rows_s, rows_r, rows_o,
                gsems, osems, isem):
    wid = lax.axis_index("s") * NC + lax.axis_index("c")
    base = wid * EPT

    pltpu.async_copy(snd3_hbm.at[wid], idx_s2, isem).wait()
    pltpu.async_copy(rcv3_hbm.at[wid], idx_r2, isem).wait()

    def issue_gather(i, b):
        pltpu.async_copy(sp_hbm.at[idx_s2.at[i]], rows_s.at[b], gsems[b])
        pltpu.async_copy(rp_hbm.at[idx_r2.at[i]], rows_r.at[b], gsems[b])

    def wait_gather(i, b):
        pltpu.make_async_copy(sp_hbm.at[idx_s2.at[i]], rows_s.at[b], gsems[b]).wait()
        pltpu.make_async_copy(rp_hbm.at[idx_r2.at[i]], rows_r.at[b], gsems[b]).wait()

    def process(i, b, prefetch):
        if prefetch:
            issue_gather(i + (GBUF - 1), (b + GBUF - 1) % GBUF)
        wait_gather(i, b)

        @pl.when(i >= GBUF)
        def _():
            pltpu.make_async_copy(rows_o.at[b], out_hbm.at[pl.ds(base, KC)],
                                  osems[b]).wait()

        def add_row(j, c2):
            for l in range(D // 16):
                sl = pl.ds(l * 16, 16)
                rows_o[b, j, sl] = rows_s[b, j, sl] + rows_r[b, j, sl]
            return c2

        lax.fori_loop(0, KC, add_row, 0, unroll=8)
        pltpu.async_copy(rows_o.at[b], out_hbm.at[pl.ds(base + i * KC, KC)],
                         osems[b])

    for b in range(GBUF - 1):
        issue_gather(b, b)

    @pl.loop(0, GMAIN, step=GBUF)
    def _(g):
        for b in range(GBUF):
            process(g + b, b, True)

    for i in range(GMAIN, NCH):
        process(i, i % GBUF, i + (GBUF - 1) < NCH)

    for b in range(GBUF):
        pltpu.make_async_copy(rows_o.at[b], out_hbm.at[pl.ds(base, KC)],
                              osems[b]).wait()


def _edge_body(gs_ref, el_ref, w1_ref, b1_ref, w2_ref, b2_ref, g_ref, bt_ref,
               alias_ref, me_ref, out_ref):
    del alias_ref
    el = el_ref[...]
    h1 = gs_ref[...] + jnp.dot(el, w1_ref[...], preferred_element_type=jnp.float32)
    h1 = jnp.maximum(h1 + b1_ref[...], 0.0)
    h2 = jnp.dot(h1, w2_ref[...], preferred_element_type=jnp.float32)
    h2 = jnp.maximum(h2 + b2_ref[...], 0.0)
    mu = jnp.mean(h2, axis=-1, keepdims=True)
    var = jnp.mean((h2 - mu) ** 2, axis=-1, keepdims=True)
    ln = (h2 - mu) * lax.rsqrt(var + 1e-5) * g_ref[...] + bt_ref[...]
    me_ref[...] = ln
    out_ref[...] = ln + el


_BE = 2000
_NBLK = ES // _BE


def _edge_mlp(sl, gsum_s, edge_latents, w1c, b1, w2, b2, g, bt, out_carry):
    row = lambda i: (i, 0)
    off_row = lambda i, sl=sl: (sl * _NBLK + i, 0)
    full = lambda i: (0, 0)
    args = [gsum_s, edge_latents, w1c, b1, w2, b2, g, bt]
    in_specs = [
        pl.BlockSpec((_BE, D), row),
        pl.BlockSpec((_BE, D), off_row),
        pl.BlockSpec((D, D), full),
        pl.BlockSpec((1, D), full),
        pl.BlockSpec((D, D), full),
        pl.BlockSpec((1, D), full),
        pl.BlockSpec((1, D), full),
        pl.BlockSpec((1, D), full),
        pl.BlockSpec(memory_space=pl.ANY),
    ]
    if out_carry is None:
        out_carry = jnp.zeros((8, D), jnp.float32)
        in_specs[-1] = pl.BlockSpec(memory_space=pl.ANY)
        io_alias = {}
    else:
        io_alias = {8: 1}
    return pl.pallas_call(
        _edge_body,
        grid=(_NBLK,),
        in_specs=in_specs,
        out_specs=[
            pl.BlockSpec((_BE, D), row),
            pl.BlockSpec((_BE, D), off_row),
        ],
        out_shape=[
            jax.ShapeDtypeStruct((ES, D), jnp.float32),
            jax.ShapeDtypeStruct((E, D), jnp.float32),
        ],
        input_output_aliases=io_alias,
    )(*args, out_carry)


def _make_scatter(slices):
    n_mes = len(slices)

    @functools.partial(
        pl.kernel,
        out_type=jax.ShapeDtypeStruct((NC, NPAD, D), jnp.float32),
        mesh=_MESH,
        scratch_types=[
            pltpu.VMEM((NCHP, KC), jnp.int32),
            pltpu.VMEM((NBUF, KC, D), jnp.float32),
            pltpu.VMEM_SHARED((NPAD, D), jnp.float32),
            [pltpu.SemaphoreType.DMA] * NBUF,
            [pltpu.SemaphoreType.DMA] * NBUF,
            pltpu.SemaphoreType.DMA,
        ],
    )
    def _scatter(*refs):
        mes = refs[:n_mes]
        rcv4_hbm, zeros_hbm, out_hbm, idx2, rows, acc, rsems, ssems, isem = refs[n_mes:]
        _scatter_body(slices, mes, rcv4_hbm, zeros_hbm, out_hbm,
                      idx2, rows, acc, rsems, ssems, isem)

    return _scatter


def _scatter_body(slices, mes, rcv4_hbm, zeros_hbm, out_hbm,
                  idx2, rows, acc, rsems, ssems, isem):
    c = lax.axis_index("c")
    s = lax.axis_index("s")
    wid = s * NC + c
    rbase = s * ROWS_PT
    base = wid * EPT

    zcp = pltpu.async_copy(zeros_hbm.at[pl.ds(rbase, ROWS_PT)],
                           acc.at[pl.ds(rbase, ROWS_PT)], isem)
    zcp.wait()
    plsc.subcore_barrier()

    for k, sl in enumerate(slices):
        me_hbm = mes[k]
        pltpu.async_copy(rcv4_hbm.at[sl, wid], idx2, isem).wait()

        def load_rows(i, b, me_hbm=me_hbm):
            pltpu.async_copy(me_hbm.at[pl.ds(base + i * KC, KC)], rows.at[b],
                             rsems[b])

        def process(i, b, prefetch, me_hbm=me_hbm, load_rows=load_rows):
            @pl.when(i >= 1)
            def _():
                pltpu.make_async_copy(rows.at[1 - b], acc.at[idx2.at[i]],
                                      ssems[1 - b]).wait()
            if prefetch:
                load_rows(i + 1, 1 - b)
            pltpu.make_async_copy(me_hbm.at[pl.ds(base, KC)], rows.at[b],
                                  rsems[b]).wait()
            pltpu.async_copy(rows.at[b], acc.at[idx2.at[i]], ssems[b], add=True)

        load_rows(0, 0)

        @pl.loop(0, NCH - 1, step=NBUF)
        def _(g):
            for b in range(NBUF):
                process(g + b, b, True)

        process(NCH - 1, (NCH - 1) % NBUF, False)
        pltpu.make_async_copy(rows.at[(NCH - 1) % NBUF], acc.at[idx2.at[0]],
                              ssems[(NCH - 1) % NBUF]).wait()

    plsc.subcore_barrier()
    pltpu.sync_copy(acc.at[pl.ds(rbase, ROWS_PT)],
                    out_hbm.at[c, pl.ds(rbase, ROWS_PT)])


_scatter_a = _make_scatter((0, 1, 2))
_scatter_b = _make_scatter((3, 4))


def _node_body(nd_ref, pa_ref, pb_ref, w1a_ref, w1b_ref, b1_ref, w2_ref, b2_ref,
               g_ref, bt_ref, out_ref):
    nd = nd_ref[...]
    aggr = (pa_ref[0] + pa_ref[1]) + (pb_ref[0] + pb_ref[1])
    h1 = (jnp.dot(nd, w1a_ref[...], preferred_element_type=jnp.float32)
          + jnp.dot(aggr, w1b_ref[...], preferred_element_type=jnp.float32))
    h1 = jnp.maximum(h1 + b1_ref[...], 0.0)
    h2 = jnp.dot(h1, w2_ref[...], preferred_element_type=jnp.float32)
    h2 = jnp.maximum(h2 + b2_ref[...], 0.0)
    mu = jnp.mean(h2, axis=-1, keepdims=True)
    var = jnp.mean((h2 - mu) ** 2, axis=-1, keepdims=True)
    ln = (h2 - mu) * lax.rsqrt(var + 1e-5) * g_ref[...] + bt_ref[...]
    out_ref[...] = ln + nd


def _node_mlp(node, pa, pb, w1a, w1b, b1, w2, b2, g, bt):
    bn = 1000
    grid = (N // bn,)
    row = lambda i: (i, 0)
    full = lambda i: (0, 0)
    return pl.pallas_call(
        _node_body,
        grid=grid,
        in_specs=[
            pl.BlockSpec((bn, D), row),
            pl.BlockSpec((NC, bn, D), lambda i: (0, i, 0)),
            pl.BlockSpec((NC, bn, D), lambda i: (0, i, 0)),
            pl.BlockSpec((D, D), full),
            pl.BlockSpec((D, D), full),
            pl.BlockSpec((1, D), full),
            pl.BlockSpec((D, D), full),
            pl.BlockSpec((1, D), full),
            pl.BlockSpec((1, D), full),
            pl.BlockSpec((1, D), full),
        ],
        out_specs=pl.BlockSpec((bn, D), row),
        out_shape=jax.ShapeDtypeStruct((N, D), jnp.float32),
    )(node, pa, pb, w1a, w1b, b1, w2, b2, g, bt)


def kernel(node_latents, mesh_edge_latents, senders, receivers,
           eW1, eb1, eW2, eb2, eg, ebt,
           nW1, nb1, nW2, nb2, ng, nbt):
    senders = senders.astype(jnp.int32)
    receivers = receivers.astype(jnp.int32)

    def to4d(x):
        x = x.reshape(S, NW, NCH, KC)
        return jnp.pad(x, ((0, 0), (0, 0), (0, NCHP - NCH), (0, 0)))

    snd4 = to4d(senders)
    rcv4 = to4d(receivers)

    sp, rp = _project(node_latents, eW1[:D], eW1[D:2 * D])

    eb1r = eb1.reshape(1, D)
    eb2r = eb2.reshape(1, D)
    egr = eg.reshape(1, D)
    ebtr = ebt.reshape(1, D)
    w1c = eW1[2 * D:]

    mes = []
    out_carry = None
    for sl in range(S):
        gsum_s = _gather_sum(sp, rp, snd4[sl], rcv4[sl])
        me_s, out_carry = _edge_mlp(sl, gsum_s, mesh_edge_latents, w1c,
                                    eb1r, eW2, eb2r, egr, ebtr, out_carry)
        mes.append(me_s)
    new_edges = out_carry

    zeros = jnp.zeros((NPAD, D), jnp.float32)
    pa = _scatter_a(mes[0], mes[1], mes[2], rcv4, zeros)
    pb = _scatter_b(mes[3], mes[4], rcv4, zeros)
    new_nodes = _node_mlp(
        node_latents, pa, pb, nW1[:D], nW1[D:],
        nb1.reshape(1, D), nW2, nb2.reshape(1, D),
        ng.reshape(1, D), nbt.reshape(1, D))
    return (new_nodes, new_edges)
